# baseline (speedup 1.0000x reference)
import functools

import jax
import jax.numpy as jnp
from jax import lax
from jax.experimental import pallas as pl
from jax.experimental.pallas import tpu as pltpu
from jax.experimental.pallas import tpu_sc as plsc

_ALPHA = 0.25
_IOU_THR = 0.5
_NEG_POS_RATIO = 3
_B, _N, _G = 16, 16384, 20
_NR, _NC = 128, 128
_NV = _N // 16


def _dense_body(gt_ref, a_ref, b_ref, conf_ref,
                loc_out, pos_out, npos_out, k_out, vbits_out,
                iou_buf):
    i = pl.program_id(0)

    @pl.when(i == 0)
    def _init():
        loc_out[...] = jnp.zeros_like(loc_out)
        pos_out[...] = jnp.zeros_like(pos_out)
        npos_out[...] = jnp.zeros_like(npos_out)

    a1 = a_ref[0]
    a2 = a_ref[1]
    a3 = a_ref[2]
    a4 = a_ref[3]
    area_a = (a3 - a1) * (a4 - a2)

    rows = lax.broadcasted_iota(jnp.int32, (_NR, _NC), 0)
    cols = lax.broadcasted_iota(jnp.int32, (_NR, _NC), 1)
    flat = rows * _NC + cols

    best_iou = jnp.full((_NR, _NC), -1.0, jnp.float32)
    mg1 = jnp.zeros((_NR, _NC), jnp.float32)
    mg2 = jnp.zeros((_NR, _NC), jnp.float32)
    mg3 = jnp.zeros((_NR, _NC), jnp.float32)
    mg4 = jnp.zeros((_NR, _NC), jnp.float32)
    force = jnp.zeros((_NR, _NC), jnp.bool_)

    rowmax = []
    for g in range(_G):
        g1 = gt_ref[i, g, 0]
        g2 = gt_ref[i, g, 1]
        g3 = gt_ref[i, g, 2]
        g4 = gt_ref[i, g, 3]
        x1 = jnp.maximum(a1, g1)
        y1 = jnp.maximum(a2, g2)
        x2 = jnp.minimum(a3, g3)
        y2 = jnp.minimum(a4, g4)
        inter = jnp.clip(x2 - x1, 0.0) * jnp.clip(y2 - y1, 0.0)
        area_g = (g3 - g1) * (g4 - g2)
        iou_g = inter / (area_a + area_g - inter + 1e-10)
        iou_buf[g] = iou_g
        better = iou_g > best_iou
        best_iou = jnp.where(better, iou_g, best_iou)
        mg1 = jnp.where(better, g1, mg1)
        mg2 = jnp.where(better, g2, mg2)
        mg3 = jnp.where(better, g3, mg3)
        mg4 = jnp.where(better, g4, mg4)
        rowmax.append(jnp.max(iou_g, axis=0))

    colmax = [jnp.max(rm) for rm in rowmax]
    rowmin = []
    for g in range(_G):
        cand = jnp.where(iou_buf[g] == colmax[g], flat, _N)
        rowmin.append(jnp.min(cand, axis=0))
    argfirst = [jnp.min(rm) for rm in rowmin]
    for g in range(_G):
        force = force | (flat == argfirst[g])

    pos = (best_iou > _IOU_THR) | force
    npos_f = jnp.sum(pos.astype(jnp.float32))
    npos_i = npos_f.astype(jnp.int32)

    b1 = b_ref[0, 0]
    b2 = b_ref[1, 0]
    b3 = b_ref[2, 0]
    b4 = b_ref[3, 0]
    x1 = jnp.maximum(b1, mg1)
    y1 = jnp.maximum(b2, mg2)
    x2 = jnp.minimum(b3, mg3)
    y2 = jnp.minimum(b4, mg4)
    inter = jnp.clip(x2 - x1, 0.0) * jnp.clip(y2 - y1, 0.0)
    area_b = (b3 - b1) * (b4 - b2)
    area_m = (mg3 - mg1) * (mg4 - mg2)
    iou_m = inter / (area_b + area_m - inter + 1e-10)
    rho2 = ((b1 + b3 - mg1 - mg3) * 0.5) ** 2 + ((b2 + b4 - mg2 - mg4) * 0.5) ** 2
    ex1 = jnp.minimum(b1, mg1)
    ey1 = jnp.minimum(b2, mg2)
    ex2 = jnp.maximum(b3, mg3)
    ey2 = jnp.maximum(b4, mg4)
    c2 = (ex2 - ex1) ** 2 + (ey2 - ey1) ** 2
    loc_all = 1.0 - iou_m + rho2 / (c2 + 1e-10)
    loc_sum = jnp.sum(jnp.where(pos, loc_all, 0.0))

    p = conf_ref[0]
    l = jnp.log(p / (1.0 - p + 1e-10))
    pf = 1.0 / (1.0 + jnp.exp(-l))
    sp = jnp.log1p(jnp.exp(-jnp.abs(l)))
    relu_l = jnp.maximum(l, 0.0)
    focal_pos = _ALPHA * (1.0 - pf) ** 2 * (relu_l - l + sp)
    focal_neg = (1.0 - _ALPHA) * pf * pf * (relu_l + sp)
    pos_loss = jnp.sum(jnp.where(pos, focal_pos, 0.0))

    v = jnp.where(pos, 0.0, focal_neg)
    k = jnp.minimum(npos_i * _NEG_POS_RATIO, _N - npos_i)

    loc_out[...] += jnp.full(loc_out.shape, loc_sum, jnp.float32)
    pos_out[...] += jnp.full(pos_out.shape, pos_loss, jnp.float32)
    npos_out[...] += jnp.full(npos_out.shape, npos_f, jnp.float32)
    k_out[...] = jnp.full(k_out.shape, k, jnp.int32)
    vbits_out[0] = lax.bitcast_convert_type(v, jnp.int32)


_sc_mesh = plsc.VectorSubcoreMesh(core_axis_name="c", subcore_axis_name="s")


@functools.partial(
    pl.kernel,
    mesh=_sc_mesh,
    out_type=jax.ShapeDtypeStruct((_B, 16), jnp.float32),
    scratch_types=[
        pltpu.VMEM((_N,), jnp.int32),
        pltpu.VMEM((2 * _N,), jnp.int32),
        pltpu.VMEM((16,), jnp.int32),
        pltpu.VMEM((16,), jnp.float32),
    ],
    compiler_params=pltpu.CompilerParams(needs_layout_passes=False),
)
def _sc_topk_sum(vbits_hbm, k_hbm, out_hbm, vb, arena, kv, ov):
    wid = lax.axis_index("s") * 2 + lax.axis_index("c")

    @pl.when(wid < _B)
    def _():
        b = wid
        pltpu.sync_copy(vbits_hbm.at[b], vb)
        pltpu.sync_copy(k_hbm.at[b], kv)
        k_sc = jnp.max(kv[...])

        one = jnp.ones((16,), jnp.int32)
        zero = jnp.zeros((16,), jnp.int32)
        lanes = jnp.arange(16, dtype=jnp.int32)

        lo = jnp.int32(0)
        hi = jnp.int32(0x7F7FFFFF)
        cnt_c = jnp.int32(_N)
        cnt_above = jnp.int32(0)
        s_base = jnp.int32(0)

        for t in range(31):
            mid = lo + ((hi - lo + 1) >> 1)
            nv = (cnt_c + 15) >> 4

            def body(j, carry, t=t, mid=mid, s_base=s_base, cnt_c=cnt_c):
                offg, offl = carry
                if t == 0:
                    x = vb[pl.ds(j * 16, 16)]
                else:
                    x = arena[pl.ds(s_base + j * 16, 16)]
                lane_ok = (j * 16 + lanes) < cnt_c
                ge = (x >= mid) & lane_ok
                lt = (x < mid) & lane_ok
                gei = jnp.where(ge, one, zero)
                lti = jnp.where(lt, one, zero)
                idxg = offg + (jnp.cumsum(gei) - gei)
                idxl = offl + (jnp.cumsum(lti) - lti)
                plsc.store_scatter(arena, [idxg], x, mask=ge)
                plsc.store_scatter(arena, [idxl], x, mask=lt)
                return (offg + plsc.all_reduce_population_count(ge),
                        offl + plsc.all_reduce_population_count(lt))

            offg, offl = lax.fori_loop(
                0, nv, body, (zero, jnp.full((16,), _N, jnp.int32)))
            cnt_g = jnp.max(offg)
            cnt_l = jnp.max(offl) - _N
            keep_g = (cnt_above + cnt_g) >= k_sc
            lo = jnp.where(keep_g, mid, lo)
            hi = jnp.where(keep_g, hi, mid - 1)
            s_base = jnp.where(keep_g, 0, _N)
            cnt_above = jnp.where(keep_g, cnt_above, cnt_above + cnt_g)
            cnt_c = jnp.where(keep_g, cnt_g, cnt_l)

        def body2(j, sacc):
            xb = vb[pl.ds(j * 16, 16)]
            m = xb > lo
            return sacc + jnp.where(m, plsc.bitcast(xb, jnp.float32), 0.0)

        sacc = lax.fori_loop(0, _NV, body2,
                             jnp.zeros((16,), jnp.float32), unroll=8)

        sum_gt = jnp.sum(sacc)
        tval = lax.bitcast_convert_type(lo, jnp.float32)
        neg = sum_gt + (k_sc - cnt_above).astype(jnp.float32) * tval
        neg = jnp.where(k_sc > 0, neg, 0.0)
        ov[...] = jnp.full((16,), neg, jnp.float32)
        pltpu.sync_copy(ov, out_hbm.at[b])


@jax.jit
def kernel(bbox_pred, conf_pred, anchors, gt_boxes):
    at = anchors.T.reshape(4, _NR, _NC)
    bt = bbox_pred.transpose(2, 0, 1).reshape(4, _B, _NR, _NC)
    conf = conf_pred.reshape(_B, _NR, _NC)

    full3d = pl.BlockSpec((4, _NR, _NC), lambda i: (0, 0, 0))
    per_b4 = pl.BlockSpec((4, 1, _NR, _NC), lambda i: (0, i, 0, 0))
    per_b = pl.BlockSpec((1, _NR, _NC), lambda i: (i, 0, 0))
    acc = pl.BlockSpec((1, _NC), lambda i: (0, 0))
    per_row = pl.BlockSpec((1, 1, _NC), lambda i: (i, 0, 0))

    loc_p, pos_p, npos_p, k_p, vbits = pl.pallas_call(
        _dense_body,
        grid=(_B,),
        in_specs=[pl.BlockSpec(memory_space=pltpu.SMEM), full3d, per_b4, per_b],
        out_specs=[acc, acc, acc, per_row, per_b],
        out_shape=[
            jax.ShapeDtypeStruct((1, _NC), jnp.float32),
            jax.ShapeDtypeStruct((1, _NC), jnp.float32),
            jax.ShapeDtypeStruct((1, _NC), jnp.float32),
            jax.ShapeDtypeStruct((_B, 1, _NC), jnp.int32),
            jax.ShapeDtypeStruct((_B, _NR, _NC), jnp.int32),
        ],
        scratch_shapes=[pltpu.VMEM((_G, _NR, _NC), jnp.float32)],
    )(gt_boxes, at, bt, conf)

    neg_rows = _sc_topk_sum(vbits.reshape(_B, _N), k_p[:, 0, :16])

    num_pos = npos_p[0, 0].astype(jnp.int32)
    denom = jnp.maximum(1, num_pos)
    total_loc = loc_p[0, 0] / denom
    total_conf = (pos_p[0, 0] + jnp.sum(neg_rows[:, 0])) / denom
    total = 1.5 * total_loc + total_conf
    return (total, total_conf, total_loc)


# speedup vs baseline: 1.6653x; 1.6653x over previous
import functools

import jax
import jax.numpy as jnp
from jax import lax
from jax.experimental import pallas as pl
from jax.experimental.pallas import tpu as pltpu
from jax.experimental.pallas import tpu_sc as plsc

_ALPHA = 0.25
_IOU_THR = 0.5
_NEG_POS_RATIO = 3
_B, _N, _G = 16, 16384, 20
_NR, _NC = 128, 128
_NV = _N // 16


def _dense_body(gt_ref, a_ref, b_ref, conf_ref,
                loc_out, pos_out, npos_out, k_out, vbits_out,
                iou_buf):
    i = pl.program_id(0)

    @pl.when(i == 0)
    def _init():
        loc_out[...] = jnp.zeros_like(loc_out)
        pos_out[...] = jnp.zeros_like(pos_out)
        npos_out[...] = jnp.zeros_like(npos_out)

    a1 = a_ref[0]
    a2 = a_ref[1]
    a3 = a_ref[2]
    a4 = a_ref[3]
    area_a = (a3 - a1) * (a4 - a2)

    rows = lax.broadcasted_iota(jnp.int32, (_NR, _NC), 0)
    cols = lax.broadcasted_iota(jnp.int32, (_NR, _NC), 1)
    flat = rows * _NC + cols

    best_iou = jnp.full((_NR, _NC), -1.0, jnp.float32)
    mg1 = jnp.zeros((_NR, _NC), jnp.float32)
    mg2 = jnp.zeros((_NR, _NC), jnp.float32)
    mg3 = jnp.zeros((_NR, _NC), jnp.float32)
    mg4 = jnp.zeros((_NR, _NC), jnp.float32)
    force = jnp.zeros((_NR, _NC), jnp.bool_)

    rowmax = []
    for g in range(_G):
        g1 = gt_ref[i, g, 0]
        g2 = gt_ref[i, g, 1]
        g3 = gt_ref[i, g, 2]
        g4 = gt_ref[i, g, 3]
        x1 = jnp.maximum(a1, g1)
        y1 = jnp.maximum(a2, g2)
        x2 = jnp.minimum(a3, g3)
        y2 = jnp.minimum(a4, g4)
        inter = jnp.clip(x2 - x1, 0.0) * jnp.clip(y2 - y1, 0.0)
        area_g = (g3 - g1) * (g4 - g2)
        iou_g = inter / (area_a + area_g - inter + 1e-10)
        iou_buf[g] = iou_g
        better = iou_g > best_iou
        best_iou = jnp.where(better, iou_g, best_iou)
        mg1 = jnp.where(better, g1, mg1)
        mg2 = jnp.where(better, g2, mg2)
        mg3 = jnp.where(better, g3, mg3)
        mg4 = jnp.where(better, g4, mg4)
        rowmax.append(jnp.max(iou_g, axis=0))

    colmax = [jnp.max(rm) for rm in rowmax]
    rowmin = []
    for g in range(_G):
        cand = jnp.where(iou_buf[g] == colmax[g], flat, _N)
        rowmin.append(jnp.min(cand, axis=0))
    argfirst = [jnp.min(rm) for rm in rowmin]
    for g in range(_G):
        force = force | (flat == argfirst[g])

    pos = (best_iou > _IOU_THR) | force
    npos_f = jnp.sum(pos.astype(jnp.float32))
    npos_i = npos_f.astype(jnp.int32)

    b1 = b_ref[0, 0]
    b2 = b_ref[1, 0]
    b3 = b_ref[2, 0]
    b4 = b_ref[3, 0]
    x1 = jnp.maximum(b1, mg1)
    y1 = jnp.maximum(b2, mg2)
    x2 = jnp.minimum(b3, mg3)
    y2 = jnp.minimum(b4, mg4)
    inter = jnp.clip(x2 - x1, 0.0) * jnp.clip(y2 - y1, 0.0)
    area_b = (b3 - b1) * (b4 - b2)
    area_m = (mg3 - mg1) * (mg4 - mg2)
    iou_m = inter / (area_b + area_m - inter + 1e-10)
    rho2 = ((b1 + b3 - mg1 - mg3) * 0.5) ** 2 + ((b2 + b4 - mg2 - mg4) * 0.5) ** 2
    ex1 = jnp.minimum(b1, mg1)
    ey1 = jnp.minimum(b2, mg2)
    ex2 = jnp.maximum(b3, mg3)
    ey2 = jnp.maximum(b4, mg4)
    c2 = (ex2 - ex1) ** 2 + (ey2 - ey1) ** 2
    loc_all = 1.0 - iou_m + rho2 / (c2 + 1e-10)
    loc_sum = jnp.sum(jnp.where(pos, loc_all, 0.0))

    p = conf_ref[0]
    l = jnp.log(p / (1.0 - p + 1e-10))
    pf = 1.0 / (1.0 + jnp.exp(-l))
    sp = jnp.log1p(jnp.exp(-jnp.abs(l)))
    relu_l = jnp.maximum(l, 0.0)
    focal_pos = _ALPHA * (1.0 - pf) ** 2 * (relu_l - l + sp)
    focal_neg = (1.0 - _ALPHA) * pf * pf * (relu_l + sp)
    pos_loss = jnp.sum(jnp.where(pos, focal_pos, 0.0))

    v = jnp.where(pos, 0.0, focal_neg)
    k = jnp.minimum(npos_i * _NEG_POS_RATIO, _N - npos_i)

    loc_out[...] += jnp.full(loc_out.shape, loc_sum, jnp.float32)
    pos_out[...] += jnp.full(pos_out.shape, pos_loss, jnp.float32)
    npos_out[...] += jnp.full(npos_out.shape, npos_f, jnp.float32)
    k_out[...] = jnp.full(k_out.shape, k, jnp.int32)
    vbits_out[0] = lax.bitcast_convert_type(v, jnp.int32)


_sc_mesh = plsc.VectorSubcoreMesh(core_axis_name="c", subcore_axis_name="s")


@functools.partial(
    pl.kernel,
    mesh=_sc_mesh,
    out_type=jax.ShapeDtypeStruct((_B, 16), jnp.float32),
    scratch_types=[
        pltpu.VMEM((_N,), jnp.int32),
        pltpu.VMEM((16,), jnp.int32),
        pltpu.VMEM((16,), jnp.float32),
    ],
    compiler_params=pltpu.CompilerParams(needs_layout_passes=False),
)
def _sc_topk_sum(vbits_hbm, k_hbm, out_hbm, vb, kv, ov):
    wid = lax.axis_index("s") * 2 + lax.axis_index("c")

    @pl.when(wid < _B)
    def _():
        b = wid
        pltpu.sync_copy(vbits_hbm.at[b], vb)
        pltpu.sync_copy(k_hbm.at[b], kv)
        k_sc = jnp.max(kv[...])

        one = jnp.ones((16,), jnp.int32)
        zero = jnp.zeros((16,), jnp.int32)

        def count_ge(mid):
            def body(j, acc):
                m = vb[pl.ds(j * 16, 16)] >= mid
                return acc + jnp.where(m, one, zero)
            return jnp.sum(lax.fori_loop(0, _NV, body, zero, unroll=8))

        def bs(_, carry):
            lo, hi = carry
            mid = lo + ((hi - lo + 1) >> 1)
            take = count_ge(mid) >= k_sc
            return (jnp.where(take, mid, lo), jnp.where(take, hi, mid - 1))

        lo, _hi = lax.fori_loop(
            0, 31, bs, (jnp.int32(0), jnp.int32(0x7F7FFFFF)))

        def body2(j, carry):
            sacc, cacc = carry
            xb = vb[pl.ds(j * 16, 16)]
            m = xb > lo
            xf = plsc.bitcast(xb, jnp.float32)
            return (sacc + jnp.where(m, xf, 0.0),
                    cacc + jnp.where(m, one, zero))

        sacc, cacc = lax.fori_loop(
            0, _NV, body2,
            (jnp.zeros((16,), jnp.float32), zero),
            unroll=8)

        sum_gt = jnp.sum(sacc)
        cnt_gt = jnp.sum(cacc)
        tval = lax.bitcast_convert_type(lo, jnp.float32)
        neg = sum_gt + (k_sc - cnt_gt).astype(jnp.float32) * tval
        neg = jnp.where(k_sc > 0, neg, 0.0)
        ov[...] = jnp.full((16,), neg, jnp.float32)
        pltpu.sync_copy(ov, out_hbm.at[b])


@jax.jit
def kernel(bbox_pred, conf_pred, anchors, gt_boxes):
    at = anchors.T.reshape(4, _NR, _NC)
    bt = bbox_pred.transpose(2, 0, 1).reshape(4, _B, _NR, _NC)
    conf = conf_pred.reshape(_B, _NR, _NC)

    full3d = pl.BlockSpec((4, _NR, _NC), lambda i: (0, 0, 0))
    per_b4 = pl.BlockSpec((4, 1, _NR, _NC), lambda i: (0, i, 0, 0))
    per_b = pl.BlockSpec((1, _NR, _NC), lambda i: (i, 0, 0))
    acc = pl.BlockSpec((1, _NC), lambda i: (0, 0))
    per_row = pl.BlockSpec((1, 1, _NC), lambda i: (i, 0, 0))

    loc_p, pos_p, npos_p, k_p, vbits = pl.pallas_call(
        _dense_body,
        grid=(_B,),
        in_specs=[pl.BlockSpec(memory_space=pltpu.SMEM), full3d, per_b4, per_b],
        out_specs=[acc, acc, acc, per_row, per_b],
        out_shape=[
            jax.ShapeDtypeStruct((1, _NC), jnp.float32),
            jax.ShapeDtypeStruct((1, _NC), jnp.float32),
            jax.ShapeDtypeStruct((1, _NC), jnp.float32),
            jax.ShapeDtypeStruct((_B, 1, _NC), jnp.int32),
            jax.ShapeDtypeStruct((_B, _NR, _NC), jnp.int32),
        ],
        scratch_shapes=[pltpu.VMEM((_G, _NR, _NC), jnp.float32)],
    )(gt_boxes, at, bt, conf)

    neg_rows = _sc_topk_sum(vbits.reshape(_B, _N), k_p[:, 0, :16])

    num_pos = npos_p[0, 0].astype(jnp.int32)
    denom = jnp.maximum(1, num_pos)
    total_loc = loc_p[0, 0] / denom
    total_conf = (pos_p[0, 0] + jnp.sum(neg_rows[:, 0])) / denom
    total = 1.5 * total_loc + total_conf
    return (total, total_conf, total_loc)


# speedup vs baseline: 1.6933x; 1.0168x over previous
import functools

import jax
import jax.numpy as jnp
from jax import lax
from jax.experimental import pallas as pl
from jax.experimental.pallas import tpu as pltpu
from jax.experimental.pallas import tpu_sc as plsc

_ALPHA = 0.25
_IOU_THR = 0.5
_NEG_POS_RATIO = 3
_B, _N, _G = 16, 16384, 20
_NR, _NC = 128, 128
_NV = _N // 16


_BPS = 2


def _dense_body(gt_ref, a_ref, b_ref, conf_ref,
                loc_out, pos_out, npos_out, k_out, vbits_out,
                iou_buf):
    i = pl.program_id(0)

    @pl.when(i == 0)
    def _init():
        loc_out[...] = jnp.zeros_like(loc_out)
        pos_out[...] = jnp.zeros_like(pos_out)
        npos_out[...] = jnp.zeros_like(npos_out)

    a1 = a_ref[0]
    a2 = a_ref[1]
    a3 = a_ref[2]
    a4 = a_ref[3]
    area_a = (a3 - a1) * (a4 - a2)

    rows = lax.broadcasted_iota(jnp.int32, (_NR, _NC), 0)
    cols = lax.broadcasted_iota(jnp.int32, (_NR, _NC), 1)
    flat = rows * _NC + cols

    for sub in range(_BPS):
        _dense_batch(gt_ref, i * _BPS + sub, sub, area_a, flat,
                     a1, a2, a3, a4, b_ref, conf_ref,
                     loc_out, pos_out, npos_out, k_out, vbits_out, iou_buf)


def _dense_batch(gt_ref, bidx, sub, area_a, flat, a1, a2, a3, a4,
                 b_ref, conf_ref,
                 loc_out, pos_out, npos_out, k_out, vbits_out, iou_buf):
    i = bidx

    best_iou = jnp.full((_NR, _NC), -1.0, jnp.float32)
    mg1 = jnp.zeros((_NR, _NC), jnp.float32)
    mg2 = jnp.zeros((_NR, _NC), jnp.float32)
    mg3 = jnp.zeros((_NR, _NC), jnp.float32)
    mg4 = jnp.zeros((_NR, _NC), jnp.float32)
    force = jnp.zeros((_NR, _NC), jnp.bool_)

    rowmax = []
    for g in range(_G):
        g1 = gt_ref[i, g, 0]
        g2 = gt_ref[i, g, 1]
        g3 = gt_ref[i, g, 2]
        g4 = gt_ref[i, g, 3]
        x1 = jnp.maximum(a1, g1)
        y1 = jnp.maximum(a2, g2)
        x2 = jnp.minimum(a3, g3)
        y2 = jnp.minimum(a4, g4)
        inter = jnp.clip(x2 - x1, 0.0) * jnp.clip(y2 - y1, 0.0)
        area_g = (g3 - g1) * (g4 - g2)
        iou_g = inter / (area_a + area_g - inter + 1e-10)
        iou_buf[g] = iou_g
        better = iou_g > best_iou
        best_iou = jnp.where(better, iou_g, best_iou)
        mg1 = jnp.where(better, g1, mg1)
        mg2 = jnp.where(better, g2, mg2)
        mg3 = jnp.where(better, g3, mg3)
        mg4 = jnp.where(better, g4, mg4)
        rowmax.append(jnp.max(iou_g, axis=0))

    colmax = [jnp.max(rm) for rm in rowmax]
    rowmin = []
    for g in range(_G):
        cand = jnp.where(iou_buf[g] == colmax[g], flat, _N)
        rowmin.append(jnp.min(cand, axis=0))
    argfirst = [jnp.min(rm) for rm in rowmin]
    for g in range(_G):
        force = force | (flat == argfirst[g])

    pos = (best_iou > _IOU_THR) | force
    npos_f = jnp.sum(pos.astype(jnp.float32))
    npos_i = npos_f.astype(jnp.int32)

    b1 = b_ref[0, sub]
    b2 = b_ref[1, sub]
    b3 = b_ref[2, sub]
    b4 = b_ref[3, sub]
    x1 = jnp.maximum(b1, mg1)
    y1 = jnp.maximum(b2, mg2)
    x2 = jnp.minimum(b3, mg3)
    y2 = jnp.minimum(b4, mg4)
    inter = jnp.clip(x2 - x1, 0.0) * jnp.clip(y2 - y1, 0.0)
    area_b = (b3 - b1) * (b4 - b2)
    area_m = (mg3 - mg1) * (mg4 - mg2)
    iou_m = inter / (area_b + area_m - inter + 1e-10)
    rho2 = ((b1 + b3 - mg1 - mg3) * 0.5) ** 2 + ((b2 + b4 - mg2 - mg4) * 0.5) ** 2
    ex1 = jnp.minimum(b1, mg1)
    ey1 = jnp.minimum(b2, mg2)
    ex2 = jnp.maximum(b3, mg3)
    ey2 = jnp.maximum(b4, mg4)
    c2 = (ex2 - ex1) ** 2 + (ey2 - ey1) ** 2
    loc_all = 1.0 - iou_m + rho2 / (c2 + 1e-10)
    loc_sum = jnp.sum(jnp.where(pos, loc_all, 0.0))

    p = conf_ref[sub]
    l = jnp.log(p / (1.0 - p + 1e-10))
    pf = 1.0 / (1.0 + jnp.exp(-l))
    sp = jnp.log1p(jnp.exp(-jnp.abs(l)))
    relu_l = jnp.maximum(l, 0.0)
    focal_pos = _ALPHA * (1.0 - pf) ** 2 * (relu_l - l + sp)
    focal_neg = (1.0 - _ALPHA) * pf * pf * (relu_l + sp)
    pos_loss = jnp.sum(jnp.where(pos, focal_pos, 0.0))

    v = jnp.where(pos, 0.0, focal_neg)
    k = jnp.minimum(npos_i * _NEG_POS_RATIO, _N - npos_i)

    loc_out[...] += jnp.full(loc_out.shape, loc_sum, jnp.float32)
    pos_out[...] += jnp.full(pos_out.shape, pos_loss, jnp.float32)
    npos_out[...] += jnp.full(npos_out.shape, npos_f, jnp.float32)
    k_out[sub] = jnp.full((1, _NC), k, jnp.int32)
    vbits_out[sub] = lax.bitcast_convert_type(v, jnp.int32)


_sc_mesh = plsc.VectorSubcoreMesh(core_axis_name="c", subcore_axis_name="s")


@functools.partial(
    pl.kernel,
    mesh=_sc_mesh,
    out_type=jax.ShapeDtypeStruct((_B, 16), jnp.float32),
    scratch_types=[
        pltpu.VMEM((_N,), jnp.int32),
        pltpu.VMEM((16,), jnp.int32),
        pltpu.VMEM((16,), jnp.float32),
    ],
    compiler_params=pltpu.CompilerParams(needs_layout_passes=False),
)
def _sc_topk_sum(vbits_hbm, k_hbm, out_hbm, vb, kv, ov):
    wid = lax.axis_index("s") * 2 + lax.axis_index("c")

    @pl.when(wid < _B)
    def _():
        b = wid
        pltpu.sync_copy(vbits_hbm.at[b], vb)
        pltpu.sync_copy(k_hbm.at[b], kv)
        k_sc = jnp.max(kv[...])

        one = jnp.ones((16,), jnp.int32)
        zero = jnp.zeros((16,), jnp.int32)

        def count_ge(mid):
            def body(j, acc):
                m = vb[pl.ds(j * 16, 16)] >= mid
                return acc + jnp.where(m, one, zero)
            return jnp.sum(lax.fori_loop(0, _NV, body, zero, unroll=8))

        def bs(_, carry):
            lo, hi = carry
            mid = lo + ((hi - lo + 1) >> 1)
            take = count_ge(mid) >= k_sc
            return (jnp.where(take, mid, lo), jnp.where(take, hi, mid - 1))

        lo, _hi = lax.fori_loop(
            0, 31, bs, (jnp.int32(0), jnp.int32(0x7F7FFFFF)))

        def body2(j, carry):
            sacc, cacc = carry
            xb = vb[pl.ds(j * 16, 16)]
            m = xb > lo
            xf = plsc.bitcast(xb, jnp.float32)
            return (sacc + jnp.where(m, xf, 0.0),
                    cacc + jnp.where(m, one, zero))

        sacc, cacc = lax.fori_loop(
            0, _NV, body2,
            (jnp.zeros((16,), jnp.float32), zero),
            unroll=8)

        sum_gt = jnp.sum(sacc)
        cnt_gt = jnp.sum(cacc)
        tval = lax.bitcast_convert_type(lo, jnp.float32)
        neg = sum_gt + (k_sc - cnt_gt).astype(jnp.float32) * tval
        neg = jnp.where(k_sc > 0, neg, 0.0)
        ov[...] = jnp.full((16,), neg, jnp.float32)
        pltpu.sync_copy(ov, out_hbm.at[b])


@jax.jit
def kernel(bbox_pred, conf_pred, anchors, gt_boxes):
    at = anchors.T.reshape(4, _NR, _NC)
    bt = bbox_pred.transpose(2, 0, 1).reshape(4, _B, _NR, _NC)
    conf = conf_pred.reshape(_B, _NR, _NC)

    full3d = pl.BlockSpec((4, _NR, _NC), lambda i: (0, 0, 0))
    per_b4 = pl.BlockSpec((4, _BPS, _NR, _NC), lambda i: (0, i, 0, 0))
    per_b = pl.BlockSpec((_BPS, _NR, _NC), lambda i: (i, 0, 0))
    acc = pl.BlockSpec((1, _NC), lambda i: (0, 0))
    per_row = pl.BlockSpec((_BPS, 1, _NC), lambda i: (i, 0, 0))

    loc_p, pos_p, npos_p, k_p, vbits = pl.pallas_call(
        _dense_body,
        grid=(_B // _BPS,),
        in_specs=[pl.BlockSpec(memory_space=pltpu.SMEM), full3d, per_b4, per_b],
        out_specs=[acc, acc, acc, per_row, per_b],
        out_shape=[
            jax.ShapeDtypeStruct((1, _NC), jnp.float32),
            jax.ShapeDtypeStruct((1, _NC), jnp.float32),
            jax.ShapeDtypeStruct((1, _NC), jnp.float32),
            jax.ShapeDtypeStruct((_B, 1, _NC), jnp.int32),
            jax.ShapeDtypeStruct((_B, _NR, _NC), jnp.int32),
        ],
        scratch_shapes=[pltpu.VMEM((_G, _NR, _NC), jnp.float32)],
    )(gt_boxes, at, bt, conf)

    neg_rows = _sc_topk_sum(vbits.reshape(_B, _N), k_p[:, 0, :16])

    num_pos = npos_p[0, 0].astype(jnp.int32)
    denom = jnp.maximum(1, num_pos)
    total_loc = loc_p[0, 0] / denom
    total_conf = (pos_p[0, 0] + jnp.sum(neg_rows[:, 0])) / denom
    total = 1.5 * total_loc + total_conf
    return (total, total_conf, total_loc)


# speedup vs baseline: 1.6940x; 1.0004x over previous
import functools

import jax
import jax.numpy as jnp
from jax import lax
from jax.experimental import pallas as pl
from jax.experimental.pallas import tpu as pltpu
from jax.experimental.pallas import tpu_sc as plsc

_ALPHA = 0.25
_IOU_THR = 0.5
_NEG_POS_RATIO = 3
_B, _N, _G = 16, 16384, 20
_NR, _NC = 128, 128
_NV = _N // 16


_BPS = 2


def _dense_body(gt_ref, a_ref, b_ref, conf_ref,
                loc_out, pos_out, npos_out, k_out, vbits_out,
                iou_buf):
    i = pl.program_id(0)

    @pl.when(i == 0)
    def _init():
        loc_out[...] = jnp.zeros_like(loc_out)
        pos_out[...] = jnp.zeros_like(pos_out)
        npos_out[...] = jnp.zeros_like(npos_out)

    a1 = a_ref[0]
    a2 = a_ref[1]
    a3 = a_ref[2]
    a4 = a_ref[3]
    area_a = (a3 - a1) * (a4 - a2)

    rows = lax.broadcasted_iota(jnp.int32, (_NR, _NC), 0)
    cols = lax.broadcasted_iota(jnp.int32, (_NR, _NC), 1)
    flat = rows * _NC + cols

    for sub in range(_BPS):
        _dense_batch(gt_ref, i * _BPS + sub, sub, area_a, flat,
                     a1, a2, a3, a4, b_ref, conf_ref,
                     loc_out, pos_out, npos_out, k_out, vbits_out, iou_buf)


def _dense_batch(gt_ref, bidx, sub, area_a, flat, a1, a2, a3, a4,
                 b_ref, conf_ref,
                 loc_out, pos_out, npos_out, k_out, vbits_out, iou_buf):
    i = bidx

    best_iou = jnp.full((_NR, _NC), -1.0, jnp.float32)
    mg1 = jnp.zeros((_NR, _NC), jnp.float32)
    mg2 = jnp.zeros((_NR, _NC), jnp.float32)
    mg3 = jnp.zeros((_NR, _NC), jnp.float32)
    mg4 = jnp.zeros((_NR, _NC), jnp.float32)
    force = jnp.zeros((_NR, _NC), jnp.bool_)

    rowmax = []
    for g in range(_G):
        g1 = gt_ref[i, g, 0]
        g2 = gt_ref[i, g, 1]
        g3 = gt_ref[i, g, 2]
        g4 = gt_ref[i, g, 3]
        x1 = jnp.maximum(a1, g1)
        y1 = jnp.maximum(a2, g2)
        x2 = jnp.minimum(a3, g3)
        y2 = jnp.minimum(a4, g4)
        inter = jnp.clip(x2 - x1, 0.0) * jnp.clip(y2 - y1, 0.0)
        area_g = (g3 - g1) * (g4 - g2)
        iou_g = inter / (area_a + area_g - inter + 1e-10)
        iou_buf[g] = iou_g
        better = iou_g > best_iou
        best_iou = jnp.where(better, iou_g, best_iou)
        mg1 = jnp.where(better, g1, mg1)
        mg2 = jnp.where(better, g2, mg2)
        mg3 = jnp.where(better, g3, mg3)
        mg4 = jnp.where(better, g4, mg4)
        rowmax.append(jnp.max(iou_g, axis=0))

    colmax = [jnp.max(rm) for rm in rowmax]
    rowmin = []
    for g in range(_G):
        cand = jnp.where(iou_buf[g] == colmax[g], flat, _N)
        rowmin.append(jnp.min(cand, axis=0))
    argfirst = [jnp.min(rm) for rm in rowmin]
    for g in range(_G):
        force = force | (flat == argfirst[g])

    pos = (best_iou > _IOU_THR) | force
    npos_f = jnp.sum(pos.astype(jnp.float32))
    npos_i = npos_f.astype(jnp.int32)

    b1 = b_ref[0, sub]
    b2 = b_ref[1, sub]
    b3 = b_ref[2, sub]
    b4 = b_ref[3, sub]
    x1 = jnp.maximum(b1, mg1)
    y1 = jnp.maximum(b2, mg2)
    x2 = jnp.minimum(b3, mg3)
    y2 = jnp.minimum(b4, mg4)
    inter = jnp.clip(x2 - x1, 0.0) * jnp.clip(y2 - y1, 0.0)
    area_b = (b3 - b1) * (b4 - b2)
    area_m = (mg3 - mg1) * (mg4 - mg2)
    iou_m = inter / (area_b + area_m - inter + 1e-10)
    rho2 = ((b1 + b3 - mg1 - mg3) * 0.5) ** 2 + ((b2 + b4 - mg2 - mg4) * 0.5) ** 2
    ex1 = jnp.minimum(b1, mg1)
    ey1 = jnp.minimum(b2, mg2)
    ex2 = jnp.maximum(b3, mg3)
    ey2 = jnp.maximum(b4, mg4)
    c2 = (ex2 - ex1) ** 2 + (ey2 - ey1) ** 2
    loc_all = 1.0 - iou_m + rho2 / (c2 + 1e-10)
    loc_sum = jnp.sum(jnp.where(pos, loc_all, 0.0))

    p = conf_ref[sub]
    l = jnp.log(p / (1.0 - p + 1e-10))
    pf = 1.0 / (1.0 + jnp.exp(-l))
    sp = jnp.log1p(jnp.exp(-jnp.abs(l)))
    relu_l = jnp.maximum(l, 0.0)
    focal_pos = _ALPHA * (1.0 - pf) ** 2 * (relu_l - l + sp)
    focal_neg = (1.0 - _ALPHA) * pf * pf * (relu_l + sp)
    pos_loss = jnp.sum(jnp.where(pos, focal_pos, 0.0))

    v = jnp.where(pos, 0.0, focal_neg)
    k = jnp.minimum(npos_i * _NEG_POS_RATIO, _N - npos_i)

    loc_out[...] += jnp.full(loc_out.shape, loc_sum, jnp.float32)
    pos_out[...] += jnp.full(pos_out.shape, pos_loss, jnp.float32)
    npos_out[...] += jnp.full(npos_out.shape, npos_f, jnp.float32)
    k_out[sub] = jnp.full((1, _NC), k, jnp.int32)
    vbits_out[sub] = lax.bitcast_convert_type(v, jnp.int32)


_sc_mesh = plsc.VectorSubcoreMesh(core_axis_name="c", subcore_axis_name="s")


@functools.partial(
    pl.kernel,
    mesh=_sc_mesh,
    out_type=jax.ShapeDtypeStruct((_B, 16), jnp.float32),
    scratch_types=[
        pltpu.VMEM((_N,), jnp.int32),
        pltpu.VMEM((16,), jnp.int32),
        pltpu.VMEM((16,), jnp.float32),
    ],
    compiler_params=pltpu.CompilerParams(needs_layout_passes=False),
)
def _sc_topk_sum(vbits_hbm, k_hbm, out_hbm, vb, kv, ov):
    wid = lax.axis_index("s") * 2 + lax.axis_index("c")

    @pl.when(wid < _B)
    def _():
        b = wid
        pltpu.sync_copy(vbits_hbm.at[b], vb)
        pltpu.sync_copy(k_hbm.at[b], kv)
        k_sc = jnp.max(kv[...])

        one = jnp.ones((16,), jnp.int32)
        zero = jnp.zeros((16,), jnp.int32)

        def count_ge(mid):
            def body(j, acc):
                m = vb[pl.ds(j * 16, 16)] >= mid
                return acc + plsc.all_reduce_population_count(m)
            return jnp.max(lax.fori_loop(0, _NV, body, zero, unroll=16))

        def bs(_, carry):
            lo, hi = carry
            mid = lo + ((hi - lo + 1) >> 1)
            take = count_ge(mid) >= k_sc
            return (jnp.where(take, mid, lo), jnp.where(take, hi, mid - 1))

        lo, _hi = lax.fori_loop(
            0, 31, bs, (jnp.int32(0), jnp.int32(0x7F7FFFFF)))

        def body2(j, carry):
            sacc, cacc = carry
            xb = vb[pl.ds(j * 16, 16)]
            m = xb > lo
            xf = plsc.bitcast(xb, jnp.float32)
            return (sacc + jnp.where(m, xf, 0.0),
                    cacc + plsc.all_reduce_population_count(m))

        sacc, cacc = lax.fori_loop(
            0, _NV, body2,
            (jnp.zeros((16,), jnp.float32), zero),
            unroll=8)

        sum_gt = jnp.sum(sacc)
        cnt_gt = jnp.max(cacc)
        tval = lax.bitcast_convert_type(lo, jnp.float32)
        neg = sum_gt + (k_sc - cnt_gt).astype(jnp.float32) * tval
        neg = jnp.where(k_sc > 0, neg, 0.0)
        ov[...] = jnp.full((16,), neg, jnp.float32)
        pltpu.sync_copy(ov, out_hbm.at[b])


@jax.jit
def kernel(bbox_pred, conf_pred, anchors, gt_boxes):
    at = anchors.T.reshape(4, _NR, _NC)
    bt = bbox_pred.transpose(2, 0, 1).reshape(4, _B, _NR, _NC)
    conf = conf_pred.reshape(_B, _NR, _NC)

    full3d = pl.BlockSpec((4, _NR, _NC), lambda i: (0, 0, 0))
    per_b4 = pl.BlockSpec((4, _BPS, _NR, _NC), lambda i: (0, i, 0, 0))
    per_b = pl.BlockSpec((_BPS, _NR, _NC), lambda i: (i, 0, 0))
    acc = pl.BlockSpec((1, _NC), lambda i: (0, 0))
    per_row = pl.BlockSpec((_BPS, 1, _NC), lambda i: (i, 0, 0))

    loc_p, pos_p, npos_p, k_p, vbits = pl.pallas_call(
        _dense_body,
        grid=(_B // _BPS,),
        in_specs=[pl.BlockSpec(memory_space=pltpu.SMEM), full3d, per_b4, per_b],
        out_specs=[acc, acc, acc, per_row, per_b],
        out_shape=[
            jax.ShapeDtypeStruct((1, _NC), jnp.float32),
            jax.ShapeDtypeStruct((1, _NC), jnp.float32),
            jax.ShapeDtypeStruct((1, _NC), jnp.float32),
            jax.ShapeDtypeStruct((_B, 1, _NC), jnp.int32),
            jax.ShapeDtypeStruct((_B, _NR, _NC), jnp.int32),
        ],
        scratch_shapes=[pltpu.VMEM((_G, _NR, _NC), jnp.float32)],
    )(gt_boxes, at, bt, conf)

    neg_rows = _sc_topk_sum(vbits.reshape(_B, _N), k_p[:, 0, :16])

    num_pos = npos_p[0, 0].astype(jnp.int32)
    denom = jnp.maximum(1, num_pos)
    total_loc = loc_p[0, 0] / denom
    total_conf = (pos_p[0, 0] + jnp.sum(neg_rows[:, 0])) / denom
    total = 1.5 * total_loc + total_conf
    return (total, total_conf, total_loc)
